# baseline (speedup 1.0000x reference)
import functools

import jax
import jax.numpy as jnp
from jax import lax
from jax.experimental import pallas as pl
from jax.experimental.pallas import tpu as pltpu
from jax.experimental.pallas import tpu_sc as plsc

VOCAB = 1000000
EMBED_DIM = 64
BATCH = 4096
SEQ = 200

_NC = 2
_NS = 16
_NW = _NC * _NS

_B = BATCH * SEQ
_BPW = BATCH // _NW
_PER_W = _BPW * SEQ
_NBUF = 4


def _emb_kernel(idx_hbm, table_hbm, out_hbm, idx_v, idxt_v, rows_v, tbuf_v,
                gsems, wsems):
    wid = lax.axis_index("s") * _NC + lax.axis_index("c")
    base = wid * _PER_W

    pltpu.sync_copy(idx_hbm.at[pl.ds(base, _PER_W)], idx_v)

    iota = lax.iota(jnp.int32, 16)
    iota_seq = iota * SEQ

    @plsc.parallel_loop(0, SEQ, unroll=8)
    def _(s):
        for k in range(8):
            v = plsc.load_gather(idx_v, [iota_seq + (k * 16 * SEQ + s)])
            idxt_v[s, pl.ds(k * 16, 16)] = v

    def fire_gather(s, slot):
        pltpu.async_copy(table_hbm.at[idxt_v.at[s]], rows_v.at[slot],
                         gsems.at[slot])

    def wait_gather(s, slot):
        pltpu.make_async_copy(table_hbm.at[idxt_v.at[s]], rows_v.at[slot],
                              gsems.at[slot]).wait()

    def fire_writes(s, slot):
        for t in range(8):
            pltpu.async_copy(tbuf_v.at[slot, pl.ds(8 * t, 8)],
                             out_hbm.at[s, pl.ds(t * 256 + 8 * wid, 8)],
                             wsems.at[slot])

    def wait_writes(s, slot):
        for t in range(8):
            pltpu.make_async_copy(
                tbuf_v.at[slot, pl.ds(8 * t, 8)],
                out_hbm.at[s, pl.ds(t * 256 + 8 * wid, 8)],
                wsems.at[slot]).wait()

    rowsel = [iota + k * 16 for k in range(8)]

    def transpose_chunk(slot):
        buf = rows_v.at[slot]

        @plsc.parallel_loop(0, EMBED_DIM, unroll=16,
                            carry=jnp.zeros((16,), jnp.int32))
        def _(e, e_vec):
            for k in range(8):
                v = plsc.load_gather(buf, [rowsel[k], e_vec])
                tbuf_v[slot, e, pl.ds(k * 16, 16)] = v
            return e_vec + 1

    for b in range(_NBUF):
        fire_gather(b, b)

    @pl.loop(0, SEQ)
    def _(s):
        slot = lax.rem(s, _NBUF)
        wait_gather(s, slot)

        @pl.when(s >= _NBUF)
        def _():
            wait_writes(s - _NBUF, slot)

        transpose_chunk(slot)
        fire_writes(s, slot)

        @pl.when(s + _NBUF < SEQ)
        def _():
            fire_gather(s + _NBUF, slot)

    for b in range(_NBUF):
        s_last = SEQ - _NBUF + b
        wait_writes(s_last, lax.rem(jnp.int32(s_last), _NBUF))


@jax.jit
def _emb_lookup(idx_flat, table):
    mesh = plsc.VectorSubcoreMesh(core_axis_name="c", subcore_axis_name="s")
    run = pl.kernel(
        _emb_kernel,
        out_type=jax.ShapeDtypeStruct((SEQ, 2048, 128), jnp.float32),
        mesh=mesh,
        scratch_types=[
            pltpu.VMEM((_PER_W,), jnp.int32),
            pltpu.VMEM((SEQ, _BPW), jnp.int32),
            pltpu.VMEM((_NBUF, _BPW, EMBED_DIM), jnp.float32),
            pltpu.VMEM((_NBUF, EMBED_DIM, _BPW), jnp.float32),
            pltpu.SemaphoreType.DMA((_NBUF,)),
            pltpu.SemaphoreType.DMA((_NBUF,)),
        ],
        compiler_params=pltpu.CompilerParams(use_tc_tiling_on_sc=False,
                                             needs_layout_passes=False),
    )
    return run(idx_flat, table)


def kernel(inputs, table):
    idx_flat = inputs.reshape(_B).astype(jnp.int32)
    out = _emb_lookup(idx_flat, table)
    z = out.reshape(SEQ, 8, 32, 8, 128).transpose(2, 4, 0, 1, 3)
    return z.reshape(BATCH, SEQ, EMBED_DIM)


# speedup vs baseline: 1.1121x; 1.1121x over previous
import functools

import jax
import jax.numpy as jnp
from jax import lax
from jax.experimental import pallas as pl
from jax.experimental.pallas import tpu as pltpu
from jax.experimental.pallas import tpu_sc as plsc

VOCAB = 1000000
EMBED_DIM = 64
BATCH = 4096
SEQ = 200

_NC = 2
_NS = 16
_NW = _NC * _NS

_B = BATCH * SEQ
_CHUNK = 128
_PER_W = _B // _NW
_NCHUNK = _PER_W // _CHUNK
_NBUF = 4


def _emb_kernel(idx_hbm, table_hbm, out_hbm, idx_v, rows_v, sems):
    wid = lax.axis_index("s") * _NC + lax.axis_index("c")
    chunk0 = wid * _NCHUNK
    row0 = wid * _PER_W

    pltpu.sync_copy(idx_hbm.at[pl.ds(chunk0, _NCHUNK)], idx_v)

    def start_gather(j, slot):
        pltpu.async_copy(table_hbm.at[idx_v.at[j]], rows_v.at[slot],
                         sems.at[slot])

    def wait_gather(j, slot):
        pltpu.make_async_copy(table_hbm.at[idx_v.at[j]], rows_v.at[slot],
                              sems.at[slot]).wait()

    def write_out(j, slot):
        pltpu.sync_copy(rows_v.at[slot],
                        out_hbm.at[pl.ds(row0 + j * _CHUNK, _CHUNK)])

    for b in range(_NBUF):
        start_gather(b, b)

    @pl.loop(0, _NCHUNK - _NBUF, step=_NBUF)
    def _(g):
        for b in range(_NBUF):
            j = g + b
            wait_gather(j, b)
            write_out(j, b)
            start_gather(j + _NBUF, b)

    for b in range(_NBUF):
        j = _NCHUNK - _NBUF + b
        wait_gather(j, b)
        write_out(j, b)


@jax.jit
def _emb_lookup(idx2d, table):
    mesh = plsc.VectorSubcoreMesh(core_axis_name="c", subcore_axis_name="s")
    run = pl.kernel(
        _emb_kernel,
        out_type=jax.ShapeDtypeStruct((_B, EMBED_DIM), jnp.float32),
        mesh=mesh,
        scratch_types=[
            pltpu.VMEM((_NCHUNK, _CHUNK), jnp.int32),
            pltpu.VMEM((_NBUF, _CHUNK, EMBED_DIM), jnp.float32),
            pltpu.SemaphoreType.DMA((_NBUF,)),
        ],
        compiler_params=pltpu.CompilerParams(use_tc_tiling_on_sc=False),
    )
    return run(idx2d, table)


def kernel(inputs, table):
    idx2d = inputs.reshape(_B // _CHUNK, _CHUNK).astype(jnp.int32)
    out = _emb_lookup(idx2d, table)
    return out.reshape(BATCH, SEQ, EMBED_DIM)
